# baseline (speedup 1.0000x reference)
import functools

import jax
import jax.numpy as jnp
from jax import lax
from jax.experimental import pallas as pl
from jax.experimental.pallas import tpu as pltpu
from jax.experimental.pallas import tpu_sc as plsc

C = 10
NC = 2
NS = 16
NW = NC * NS
ROWS_W = 1024 // NW
RCH = 8
CCH = 512
NCOL = 1024 // CCH
NRC = ROWS_W // RCH
ELEMS = RCH * CCH
G = ELEMS // 16

_mesh = plsc.VectorSubcoreMesh(core_axis_name="c", subcore_axis_name="s")


@functools.partial(
    pl.kernel,
    out_type=jax.ShapeDtypeStruct((C, 1024, 1024), jnp.float32),
    mesh=_mesh,
    scratch_types=[
        [pltpu.VMEM((RCH, CCH), jnp.int32)] * 2,
        [pltpu.VMEM((C, RCH, CCH), jnp.float32)] * 2,
        [pltpu.SemaphoreType.DMA] * 2,
        [pltpu.SemaphoreType.DMA] * 2,
    ],
    compiler_params=pltpu.CompilerParams(
        needs_layout_passes=False, disable_bounds_checks=True
    ),
)
def _onehot_sc(x_hbm, zeros_hbm, out_hbm, xins, obs, insems, outsems):
    wid = lax.axis_index("s") * NC + lax.axis_index("c")
    base_row = wid * ROWS_W

    iota16 = lax.iota(jnp.int32, 16)
    ones = jnp.full((16,), 1.0, jnp.float32)
    zeros = jnp.zeros((16,), jnp.float32)

    def in_slice(i):
        rc = i // NCOL
        cc = i % NCOL
        return x_hbm.at[
            pl.ds(base_row + rc * RCH, RCH), pl.ds(cc * CCH, CCH)
        ]

    def out_slice(i):
        rc = i // NCOL
        cc = i % NCOL
        return out_hbm.at[
            :, pl.ds(base_row + rc * RCH, RCH), pl.ds(cc * CCH, CCH)
        ]

    def start_in(i, b):
        pltpu.make_async_copy(in_slice(i), xins[b], insems[b]).start()

    def wait_in(i, b):
        pltpu.make_async_copy(in_slice(i), xins[b], insems[b]).wait()

    def start_out(i, b):
        pltpu.make_async_copy(obs[b], out_slice(i), outsems[b]).start()

    def wait_out(i, b):
        pltpu.make_async_copy(obs[b], out_slice(i), outsems[b]).wait()

    def scatter_pass(b, payload):
        xf = xins[b]
        of = obs[b]
        for r in range(RCH):
            row_vec = jnp.full((16,), r, jnp.int32)

            @pl.loop(0, CCH // 16, init_carry=iota16, unroll=8)
            def _sc(j, colv):
                vals = xf[r, pl.ds(j * 16, 16)]
                plsc.store_scatter(of, [vals, row_vec, colv], payload)
                return colv + 16

    def ones_pass(b):
        scatter_pass(b, ones)

    def clean_pass(b):
        scatter_pass(b, zeros)

    pltpu.sync_copy(zeros_hbm, obs[0])
    pltpu.sync_copy(zeros_hbm, obs[1])

    for b in range(2):
        pltpu.sync_copy(in_slice(b), xins[b])
        ones_pass(b)
        start_out(b, b)

    @pl.loop(1, NRC)
    def _steady(rc):
        for b in range(2):
            i = rc * NCOL + b
            wait_out(i - 2, b)
            clean_pass(b)
            pltpu.sync_copy(in_slice(i), xins[b])
            ones_pass(b)
            start_out(i, b)

    wait_out(2 * NRC - 2, 0)
    wait_out(2 * NRC - 1, 1)


def kernel(x):
    x2d = x.reshape(1024, 1024)
    zeros_stage = jnp.zeros((C, RCH, CCH), jnp.float32)
    out_cm = _onehot_sc(x2d, zeros_stage)
    return out_cm.transpose(1, 2, 0)


# speedup vs baseline: 1.5957x; 1.5957x over previous
import functools

import jax
import jax.numpy as jnp
from jax import lax
from jax.experimental import pallas as pl
from jax.experimental.pallas import tpu as pltpu
from jax.experimental.pallas import tpu_sc as plsc

C = 10
NC = 2
NS = 16
NW = NC * NS
ROWS_W = 1024 // NW
RCH = 8
CCH = 512
NCOL = 1024 // CCH
NRC = ROWS_W // RCH
ELEMS = RCH * CCH
G = ELEMS // 16

_mesh = plsc.VectorSubcoreMesh(core_axis_name="c", subcore_axis_name="s")


@functools.partial(
    pl.kernel,
    out_type=jax.ShapeDtypeStruct((C, 1024, 1024), jnp.float32),
    mesh=_mesh,
    scratch_types=[
        [pltpu.VMEM((RCH, CCH), jnp.int32)] * 2,
        [pltpu.VMEM((C, RCH, CCH), jnp.float32)] * 2,
        [pltpu.SemaphoreType.DMA] * 2,
        [pltpu.SemaphoreType.DMA] * 2,
    ],
    compiler_params=pltpu.CompilerParams(
        needs_layout_passes=False, disable_bounds_checks=True
    ),
)
def _onehot_sc(x_hbm, zeros_hbm, out_hbm, xins, obs, insems, outsems):
    wid = lax.axis_index("s") * NC + lax.axis_index("c")
    base_row = wid * ROWS_W

    iota16 = lax.iota(jnp.int32, 16)
    ones = jnp.full((16,), 1.0, jnp.float32)
    zeros = jnp.zeros((16,), jnp.float32)

    def in_slice(i):
        rc = i // NCOL
        cc = i % NCOL
        return x_hbm.at[
            pl.ds(base_row + rc * RCH, RCH), pl.ds(cc * CCH, CCH)
        ]

    def out_slice(i):
        rc = i // NCOL
        cc = i % NCOL
        return out_hbm.at[
            :, pl.ds(base_row + rc * RCH, RCH), pl.ds(cc * CCH, CCH)
        ]

    def start_in(i, b):
        pltpu.make_async_copy(in_slice(i), xins[b], insems[b]).start()

    def wait_in(i, b):
        pltpu.make_async_copy(in_slice(i), xins[b], insems[b]).wait()

    def start_out(i, b):
        pltpu.make_async_copy(obs[b], out_slice(i), outsems[b]).start()

    def wait_out(i, b):
        pltpu.make_async_copy(obs[b], out_slice(i), outsems[b]).wait()

    def scatter_pass(b, payload):
        xf = xins[b]
        of = obs[b]
        for r in range(RCH):
            row_vec = jnp.full((16,), r, jnp.int32)

            @plsc.parallel_loop(0, CCH // 16, 1, unroll=8, carry=iota16)
            def _sc(j, colv):
                vals = xf[r, pl.ds(j * 16, 16)]
                plsc.store_scatter(of, [vals, row_vec, colv], payload)
                return colv + 16

    def ones_pass(b):
        scatter_pass(b, ones)

    def clean_pass(b):
        scatter_pass(b, zeros)

    pltpu.sync_copy(zeros_hbm, obs[0])
    pltpu.sync_copy(zeros_hbm, obs[1])

    for b in range(2):
        pltpu.sync_copy(in_slice(b), xins[b])
        ones_pass(b)
        start_out(b, b)

    @pl.loop(1, NRC)
    def _steady(rc):
        for b in range(2):
            i = rc * NCOL + b
            wait_out(i - 2, b)
            clean_pass(b)
            pltpu.sync_copy(in_slice(i), xins[b])
            ones_pass(b)
            start_out(i, b)

    wait_out(2 * NRC - 2, 0)
    wait_out(2 * NRC - 1, 1)


def kernel(x):
    x2d = x.reshape(1024, 1024)
    zeros_stage = jnp.zeros((C, RCH, CCH), jnp.float32)
    out_cm = _onehot_sc(x2d, zeros_stage)
    return out_cm.transpose(1, 2, 0)


# speedup vs baseline: 1.6821x; 1.0542x over previous
import functools

import jax
import jax.numpy as jnp
from jax import lax
from jax.experimental import pallas as pl
from jax.experimental.pallas import tpu as pltpu
from jax.experimental.pallas import tpu_sc as plsc

C = 10
NC = 2
NS = 16
NW = NC * NS
ROWS_W = 1024 // NW
RCH = 8
CCH = 512
NCOL = 1024 // CCH
NRC = ROWS_W // RCH
ELEMS = RCH * CCH
G = ELEMS // 16

_mesh = plsc.VectorSubcoreMesh(core_axis_name="c", subcore_axis_name="s")


@functools.partial(
    pl.kernel,
    out_type=jax.ShapeDtypeStruct((C, 1024, 1024), jnp.float32),
    mesh=_mesh,
    scratch_types=[
        [pltpu.VMEM((RCH, CCH), jnp.int32)] * 2,
        [pltpu.VMEM((RCH, CCH), jnp.int32)] * 2,
        [pltpu.VMEM((C, RCH, CCH), jnp.float32)] * 2,
        [pltpu.SemaphoreType.DMA] * 2,
        [pltpu.SemaphoreType.DMA] * 2,
    ],
    compiler_params=pltpu.CompilerParams(
        needs_layout_passes=False, disable_bounds_checks=True
    ),
)
def _onehot_sc(x_hbm, zeros_hbm, out_hbm, xins, vbs, obs, insems, outsems):
    wid = lax.axis_index("s") * NC + lax.axis_index("c")
    base_row = wid * ROWS_W

    iota16 = lax.iota(jnp.int32, 16)
    ones = jnp.full((16,), 1.0, jnp.float32)
    zeros = jnp.zeros((16,), jnp.float32)

    def in_slice(i):
        rc = i // NCOL
        cc = i % NCOL
        return x_hbm.at[
            pl.ds(base_row + rc * RCH, RCH), pl.ds(cc * CCH, CCH)
        ]

    def out_slice(i):
        rc = i // NCOL
        cc = i % NCOL
        return out_hbm.at[
            :, pl.ds(base_row + rc * RCH, RCH), pl.ds(cc * CCH, CCH)
        ]

    def start_in(i, b):
        pltpu.make_async_copy(in_slice(i), xins[b], insems[b]).start()

    def wait_in(i, b):
        pltpu.make_async_copy(in_slice(i), xins[b], insems[b]).wait()

    def start_out(i, b):
        pltpu.make_async_copy(obs[b], out_slice(i), outsems[b]).start()

    def wait_out(i, b):
        pltpu.make_async_copy(obs[b], out_slice(i), outsems[b]).wait()

    def ones_pass(b):
        xf = xins[b]
        vb = vbs[b]
        of = obs[b]
        for r in range(RCH):
            row_vec = jnp.full((16,), r, jnp.int32)

            @plsc.parallel_loop(0, CCH // 16, 1, unroll=8, carry=iota16)
            def _sc(j, colv):
                vals = xf[r, pl.ds(j * 16, 16)]
                vb[r, pl.ds(j * 16, 16)] = vals
                plsc.store_scatter(of, [vals, row_vec, colv], ones)
                return colv + 16

    def clean_pass(b):
        vb = vbs[b]
        of = obs[b]
        for r in range(RCH):
            row_vec = jnp.full((16,), r, jnp.int32)

            @plsc.parallel_loop(0, CCH // 16, 1, unroll=8, carry=iota16)
            def _sc(j, colv):
                vals = vb[r, pl.ds(j * 16, 16)]
                plsc.store_scatter(of, [vals, row_vec, colv], zeros)
                return colv + 16

    NCHUNK = NRC * NCOL

    start_in(0, 0)
    start_in(1, 1)
    pltpu.sync_copy(zeros_hbm, obs[0])
    pltpu.sync_copy(zeros_hbm, obs[1])

    for b in range(2):
        wait_in(b, b)
        ones_pass(b)
        start_out(b, b)
        start_in(b + 2, b)

    @pl.loop(1, NRC)
    def _steady(rc):
        for b in range(2):
            i = rc * NCOL + b
            wait_out(i - 2, b)
            clean_pass(b)
            wait_in(i, b)
            ones_pass(b)
            start_out(i, b)
            start_in(jnp.minimum(i + 2, NCHUNK - 1), b)

    wait_in(NCHUNK - 1, 0)
    wait_in(NCHUNK - 1, 1)
    wait_out(NCHUNK - 2, 0)
    wait_out(NCHUNK - 1, 1)


def kernel(x):
    x2d = x.reshape(1024, 1024)
    zeros_stage = jnp.zeros((C, RCH, CCH), jnp.float32)
    out_cm = _onehot_sc(x2d, zeros_stage)
    return out_cm.transpose(1, 2, 0)
